# baseline (speedup 1.0000x reference)
import functools

import jax
import jax.numpy as jnp
import numpy as np
from jax import lax
from jax.experimental import pallas as pl
from jax.experimental.pallas import tpu as pltpu
from jax.experimental.pallas import tpu_sc as plsc

_STRIDE = 16
_NEG_OV = 0.3
_POS_OV = 0.7
_NC = 1
_NS = 16
_NW = _NC * _NS
_L = 16


def _base_anchors(base_size=16, ratios=(0.5, 1.0, 2.0), scales=(8, 16, 32)):
    base = np.array([1, 1, base_size, base_size], dtype=np.float64) - 1
    w = base[2] - base[0] + 1
    h = base[3] - base[1] + 1
    x_ctr = base[0] + 0.5 * (w - 1)
    y_ctr = base[1] + 0.5 * (h - 1)
    size = w * h
    out = []
    for r in ratios:
        ws = np.round(np.sqrt(size / r))
        hs = np.round(ws * r)
        for s in scales:
            wss = ws * s
            hss = hs * s
            out.append([x_ctr - 0.5 * (wss - 1), y_ctr - 0.5 * (hss - 1),
                        x_ctr + 0.5 * (wss - 1), y_ctr + 0.5 * (hss - 1)])
    return np.array(out, dtype=np.float32)


def _inside_anchors(shape, stride):
    rr, cc = shape
    shift_x = np.arange(0, cc) * stride
    shift_y = np.arange(0, rr) * stride
    sx, sy = np.meshgrid(shift_x, shift_y)
    shifts = np.stack([sx.ravel(), sy.ravel(), sx.ravel(), sy.ravel()],
                      axis=1).astype(np.float32)
    base = _base_anchors(base_size=stride)
    all_anchors = (base.reshape(1, -1, 4) + shifts.reshape(-1, 1, 4)).reshape(-1, 4)
    all_anchors = all_anchors.astype(np.float32)
    mask = ((all_anchors[:, 0] >= 0) & (all_anchors[:, 1] >= 0) &
            (all_anchors[:, 2] < 800.0) & (all_anchors[:, 3] < 800.0))
    return all_anchors[np.where(mask)[0]]


def _sc_body(aw, n_gt, g_pad,
             a_hbm, gtb_hbm, gp_hbm,
             rowmax_hbm, bb_hbm, cmax_hbm, carg_hbm,
             av, gtv, gpv, area_v, rm_v, ra_v, cm_v, ca_v, ga_v,
             gw_v, gh_v, gcx_v, gcy_v, bb_v):
    wid = lax.axis_index("s") * _NC + lax.axis_index("c")
    nvec = aw // _L
    gvec = g_pad // _L
    lane = lax.broadcasted_iota(jnp.int32, (_L,), 0)

    pltpu.sync_copy(a_hbm.at[wid], av)
    pltpu.sync_copy(gtb_hbm, gtv)
    pltpu.sync_copy(gp_hbm, gpv)

    for v in range(nvec):
        sl = pl.ds(v * _L, _L)
        ax1 = av[0, sl]
        ay1 = av[1, sl]
        ax2 = av[2, sl]
        ay2 = av[3, sl]
        area_v[sl] = (ax2 - ax1 + 1.0) * (ay2 - ay1 + 1.0)
        rm_v[sl] = jnp.full((_L,), -3.0e38, jnp.float32)
        ra_v[sl] = jnp.zeros((_L,), jnp.int32)
    for v in range(gvec):
        sl = pl.ds(v * _L, _L)
        gx1 = gpv[0, sl]
        gy1 = gpv[1, sl]
        gx2 = gpv[2, sl]
        gy2 = gpv[3, sl]
        gw_v[sl] = gx2 - gx1 + 1.0
        gh_v[sl] = gy2 - gy1 + 1.0
        gcx_v[sl] = gx1 + 0.5 * (gx2 - gx1 + 1.0)
        gcy_v[sl] = gy1 + 0.5 * (gy2 - gy1 + 1.0)
        ga_v[sl] = (gx2 - gx1 + 1.0) * (gy2 - gy1 + 1.0)

    base = wid * aw

    def g_step(gp_i, _):
        g0 = gp_i * 2
        g1 = g0 + 1
        gt0 = [gtv[g0, c, :] for c in range(4)]
        gt1 = [gtv[g1, c, :] for c in range(4)]
        area0 = plsc.load_gather(ga_v, [jnp.full((_L,), g0, jnp.int32)])
        area1 = plsc.load_gather(ga_v, [jnp.full((_L,), g1, jnp.int32)])
        cmax0 = jnp.full((_L,), -3.0e38, jnp.float32)
        cvec0 = jnp.zeros((_L,), jnp.int32)
        cmax1 = jnp.full((_L,), -3.0e38, jnp.float32)
        cvec1 = jnp.zeros((_L,), jnp.int32)
        for v in range(nvec):
            sl = pl.ds(v * _L, _L)
            ax1 = av[0, sl]
            ay1 = av[1, sl]
            ax2 = av[2, sl]
            ay2 = av[3, sl]
            area_a = area_v[sl]

            def iou(gt, area_g):
                x1 = jnp.maximum(ax1, gt[0])
                y1 = jnp.maximum(ay1, gt[1])
                x2 = jnp.minimum(ax2, gt[2])
                y2 = jnp.minimum(ay2, gt[3])
                iw = jnp.maximum(x2 - x1 + 1.0, 0.0)
                ih = jnp.maximum(y2 - y1 + 1.0, 0.0)
                inter = iw * ih
                return inter / (area_a + area_g - inter)

            ov0 = iou(gt0, area0)
            ov1 = iou(gt1, area1)
            swap = ov1 > ov0
            obest = jnp.where(swap, ov1, ov0)
            gbest = jnp.where(swap, g1, g0)
            rm = rm_v[sl]
            upd = obest > rm
            rm_v[sl] = jnp.where(upd, obest, rm)
            ra_v[sl] = jnp.where(upd, gbest, ra_v[sl])
            c0 = ov0 > cmax0
            cvec0 = jnp.where(c0, v, cvec0)
            cmax0 = jnp.where(c0, ov0, cmax0)
            c1 = ov1 > cmax1
            cvec1 = jnp.where(c1, v, cvec1)
            cmax1 = jnp.where(c1, ov1, cmax1)
        cm_v[g0, :] = cmax0
        ca_v[g0, :] = base + cvec0 * _L + lane
        cm_v[g1, :] = cmax1
        ca_v[g1, :] = base + cvec1 * _L + lane
        return _

    lax.fori_loop(0, n_gt // 2, g_step, None)

    for v in range(nvec):
        sl = pl.ds(v * _L, _L)
        ra = ra_v[sl]
        scx = plsc.load_gather(gcx_v, [ra])
        scy = plsc.load_gather(gcy_v, [ra])
        sw = plsc.load_gather(gw_v, [ra])
        sh = plsc.load_gather(gh_v, [ra])
        ax1 = av[0, sl]
        ay1 = av[1, sl]
        ax2 = av[2, sl]
        ay2 = av[3, sl]
        ew = ax2 - ax1 + 1.0
        eh = ay2 - ay1 + 1.0
        ecx = ax1 + 0.5 * ew
        ecy = ay1 + 0.5 * eh
        bb_v[0, sl] = (scx - ecx) / ew
        bb_v[1, sl] = (scy - ecy) / eh
        bb_v[2, sl] = sw / ew
        bb_v[3, sl] = sh / eh

    pltpu.sync_copy(rm_v, rowmax_hbm.at[wid])
    pltpu.sync_copy(bb_v, bb_hbm.at[wid])
    pltpu.sync_copy(cm_v, cmax_hbm.at[wid])
    pltpu.sync_copy(ca_v, carg_hbm.at[wid])


def _tc_merge_body(n_valid, g_valid, rm_ref, bb_ref, cm_ref, ca_ref,
                   a_ref, meta_ref, labels_ref, bbox_ref):
    cm = cm_ref[:, :]
    m = jnp.max(cm, axis=0, keepdims=True)
    cand = jnp.where(cm == m, ca_ref[:, :], jnp.int32(2 ** 30))
    carg_i = jnp.min(cand, axis=0, keepdims=True)

    n_pad = rm_ref.shape[0]
    g_pad = cm.shape[1]
    row_i = lax.broadcasted_iota(jnp.int32, (n_pad, g_pad), 0)
    pos_gt = jnp.any(row_i == carg_i, axis=1, keepdims=True)

    rm = rm_ref[:, :]
    labels = jnp.where(rm < _NEG_OV, 0.0, -1.0)
    labels = jnp.where(pos_gt, 1.0, labels)
    labels = jnp.where(rm >= _POS_OV, 1.0, labels)

    h = meta_ref[0, 0]
    w = meta_ref[0, 1]
    inside = ((a_ref[:, 0:1] >= 0.0) & (a_ref[:, 1:2] >= 0.0) &
              (a_ref[:, 2:3] < w) & (a_ref[:, 3:4] < h))
    labels_ref[:, :] = jnp.where(inside, labels, -1.0)

    bbox_ref[:, 0:2] = bb_ref[:, 0:2]
    bbox_ref[:, 2:3] = jnp.log(bb_ref[:, 2:3])
    bbox_ref[:, 3:4] = jnp.log(bb_ref[:, 3:4])


@functools.partial(jax.jit, static_argnums=(4, 5))
def _run(anchors_w, anchors_pad, gt_boxes, metadata, n_valid, g_valid):
    aw = anchors_w.shape[2]
    n_pad = _NW * aw
    g_pad = ((g_valid + _L - 1) // _L) * _L

    gtb = jnp.broadcast_to(gt_boxes[:, :, None], (g_valid, 4, _L))
    gp = jnp.zeros((4, g_pad), jnp.float32).at[:, :g_valid].set(gt_boxes.T)

    mesh = plsc.VectorSubcoreMesh(core_axis_name="c", subcore_axis_name="s",
                                  num_cores=_NC, num_subcores=_NS)
    sc = pl.kernel(
        functools.partial(_sc_body, aw, g_valid, g_pad),
        out_type=[
            jax.ShapeDtypeStruct((_NW, aw), jnp.float32),
            jax.ShapeDtypeStruct((_NW, 4, aw), jnp.float32),
            jax.ShapeDtypeStruct((_NW, g_valid, _L), jnp.float32),
            jax.ShapeDtypeStruct((_NW, g_valid, _L), jnp.int32),
        ],
        mesh=mesh,
        compiler_params=pltpu.CompilerParams(needs_layout_passes=False),
        scratch_types=[
            pltpu.VMEM((4, aw), jnp.float32),
            pltpu.VMEM((g_valid, 4, _L), jnp.float32),
            pltpu.VMEM((4, g_pad), jnp.float32),
            pltpu.VMEM((aw,), jnp.float32),
            pltpu.VMEM((aw,), jnp.float32),
            pltpu.VMEM((aw,), jnp.int32),
            pltpu.VMEM((g_valid, _L), jnp.float32),
            pltpu.VMEM((g_valid, _L), jnp.int32),
            pltpu.VMEM((g_pad,), jnp.float32),
            pltpu.VMEM((g_pad,), jnp.float32),
            pltpu.VMEM((g_pad,), jnp.float32),
            pltpu.VMEM((g_pad,), jnp.float32),
            pltpu.VMEM((g_pad,), jnp.float32),
            pltpu.VMEM((4, aw), jnp.float32),
        ],
    )
    rowmax, bb, cmax_w, carg_w = sc(anchors_w, gtb, gp)

    rowmax = rowmax.reshape(n_pad, 1)
    bb = jnp.transpose(bb, (0, 2, 1)).reshape(n_pad, 4)
    cmax_w = jnp.transpose(cmax_w, (0, 2, 1)).reshape(_NW * _L, g_valid)
    carg_w = jnp.transpose(carg_w, (0, 2, 1)).reshape(_NW * _L, g_valid)

    labels, bbox = pl.pallas_call(
        functools.partial(_tc_merge_body, n_valid, g_valid),
        out_shape=[
            jax.ShapeDtypeStruct((n_pad, 1), jnp.float32),
            jax.ShapeDtypeStruct((n_pad, 4), jnp.float32),
        ],
        in_specs=[
            pl.BlockSpec(memory_space=pltpu.VMEM),
            pl.BlockSpec(memory_space=pltpu.VMEM),
            pl.BlockSpec(memory_space=pltpu.VMEM),
            pl.BlockSpec(memory_space=pltpu.VMEM),
            pl.BlockSpec(memory_space=pltpu.VMEM),
            pl.BlockSpec(memory_space=pltpu.SMEM),
        ],
        out_specs=[
            pl.BlockSpec(memory_space=pltpu.VMEM),
            pl.BlockSpec(memory_space=pltpu.VMEM),
        ],
    )(rowmax, bb, cmax_w, carg_w, anchors_pad, metadata)
    return labels[:n_valid, 0], bbox[:n_valid, :]


def kernel(scores, gt_boxes, metadata):
    rr, cc = scores.shape[1], scores.shape[2]
    anchors_in = _inside_anchors((rr, cc), _STRIDE)
    n_valid = anchors_in.shape[0]
    aw = ((n_valid + _NW * _L - 1) // (_NW * _L)) * _L
    n_pad = _NW * aw
    pad = np.full((n_pad - n_valid, 4), -1.0e6, dtype=np.float32)
    pad[:, 2:] += 1.0
    a_full = np.concatenate([anchors_in, pad], axis=0)
    anchors_w = jnp.asarray(
        a_full.reshape(_NW, aw, 4).transpose(0, 2, 1).copy())
    anchors_pad = jnp.asarray(a_full)
    return _run(anchors_w, anchors_pad, gt_boxes, metadata,
                n_valid, gt_boxes.shape[0])


# speedup vs baseline: 1.0445x; 1.0445x over previous
import functools

import jax
import jax.numpy as jnp
import numpy as np
from jax import lax
from jax.experimental import pallas as pl
from jax.experimental.pallas import tpu as pltpu
from jax.experimental.pallas import tpu_sc as plsc

_STRIDE = 16
_NEG_OV = 0.3
_POS_OV = 0.7
_NC = 2
_NS = 16
_NW = _NC * _NS
_L = 16


def _base_anchors(base_size=16, ratios=(0.5, 1.0, 2.0), scales=(8, 16, 32)):
    base = np.array([1, 1, base_size, base_size], dtype=np.float64) - 1
    w = base[2] - base[0] + 1
    h = base[3] - base[1] + 1
    x_ctr = base[0] + 0.5 * (w - 1)
    y_ctr = base[1] + 0.5 * (h - 1)
    size = w * h
    out = []
    for r in ratios:
        ws = np.round(np.sqrt(size / r))
        hs = np.round(ws * r)
        for s in scales:
            wss = ws * s
            hss = hs * s
            out.append([x_ctr - 0.5 * (wss - 1), y_ctr - 0.5 * (hss - 1),
                        x_ctr + 0.5 * (wss - 1), y_ctr + 0.5 * (hss - 1)])
    return np.array(out, dtype=np.float32)


def _inside_anchors(shape, stride):
    rr, cc = shape
    shift_x = np.arange(0, cc) * stride
    shift_y = np.arange(0, rr) * stride
    sx, sy = np.meshgrid(shift_x, shift_y)
    shifts = np.stack([sx.ravel(), sy.ravel(), sx.ravel(), sy.ravel()],
                      axis=1).astype(np.float32)
    base = _base_anchors(base_size=stride)
    all_anchors = (base.reshape(1, -1, 4) + shifts.reshape(-1, 1, 4)).reshape(-1, 4)
    all_anchors = all_anchors.astype(np.float32)
    mask = ((all_anchors[:, 0] >= 0) & (all_anchors[:, 1] >= 0) &
            (all_anchors[:, 2] < 800.0) & (all_anchors[:, 3] < 800.0))
    return all_anchors[np.where(mask)[0]]


def _sc_body(aw, n_gt, g_pad,
             a_hbm, gtb_hbm, gp_hbm,
             rowmax_hbm, bb_hbm, cmax_hbm, carg_hbm,
             av, gtv, gpv, area_v, rm_v, ra_v, cm_v, ca_v, ga_v,
             gw_v, gh_v, gcx_v, gcy_v, bb_v):
    wid = lax.axis_index("s") * _NC + lax.axis_index("c")
    nvec = aw // _L
    gvec = g_pad // _L
    lane = lax.broadcasted_iota(jnp.int32, (_L,), 0)

    pltpu.sync_copy(a_hbm.at[wid], av)
    pltpu.sync_copy(gtb_hbm, gtv)
    pltpu.sync_copy(gp_hbm, gpv)

    for v in range(nvec):
        sl = pl.ds(v * _L, _L)
        ax1 = av[0, sl]
        ay1 = av[1, sl]
        ax2 = av[2, sl]
        ay2 = av[3, sl]
        area_v[sl] = (ax2 - ax1 + 1.0) * (ay2 - ay1 + 1.0)
        rm_v[sl] = jnp.full((_L,), -3.0e38, jnp.float32)
        ra_v[sl] = jnp.zeros((_L,), jnp.int32)
    for v in range(gvec):
        sl = pl.ds(v * _L, _L)
        gx1 = gpv[0, sl]
        gy1 = gpv[1, sl]
        gx2 = gpv[2, sl]
        gy2 = gpv[3, sl]
        gw_v[sl] = gx2 - gx1 + 1.0
        gh_v[sl] = gy2 - gy1 + 1.0
        gcx_v[sl] = gx1 + 0.5 * (gx2 - gx1 + 1.0)
        gcy_v[sl] = gy1 + 0.5 * (gy2 - gy1 + 1.0)
        ga_v[sl] = (gx2 - gx1 + 1.0) * (gy2 - gy1 + 1.0)

    base = wid * aw

    def g_step(gp_i, _):
        g0 = gp_i * 2
        g1 = g0 + 1
        gt0 = [gtv[g0, c, :] for c in range(4)]
        gt1 = [gtv[g1, c, :] for c in range(4)]
        area0 = plsc.load_gather(ga_v, [jnp.full((_L,), g0, jnp.int32)])
        area1 = plsc.load_gather(ga_v, [jnp.full((_L,), g1, jnp.int32)])
        cmax0 = jnp.full((_L,), -3.0e38, jnp.float32)
        cvec0 = jnp.zeros((_L,), jnp.int32)
        cmax1 = jnp.full((_L,), -3.0e38, jnp.float32)
        cvec1 = jnp.zeros((_L,), jnp.int32)
        for v in range(nvec):
            sl = pl.ds(v * _L, _L)
            ax1 = av[0, sl]
            ay1 = av[1, sl]
            ax2 = av[2, sl]
            ay2 = av[3, sl]
            area_a = area_v[sl]

            def iou(gt, area_g):
                x1 = jnp.maximum(ax1, gt[0])
                y1 = jnp.maximum(ay1, gt[1])
                x2 = jnp.minimum(ax2, gt[2])
                y2 = jnp.minimum(ay2, gt[3])
                iw = jnp.maximum(x2 - x1 + 1.0, 0.0)
                ih = jnp.maximum(y2 - y1 + 1.0, 0.0)
                inter = iw * ih
                return inter / (area_a + area_g - inter)

            ov0 = iou(gt0, area0)
            ov1 = iou(gt1, area1)
            swap = ov1 > ov0
            obest = jnp.where(swap, ov1, ov0)
            gbest = jnp.where(swap, g1, g0)
            rm = rm_v[sl]
            upd = obest > rm
            rm_v[sl] = jnp.where(upd, obest, rm)
            ra_v[sl] = jnp.where(upd, gbest, ra_v[sl])
            c0 = ov0 > cmax0
            cvec0 = jnp.where(c0, v, cvec0)
            cmax0 = jnp.where(c0, ov0, cmax0)
            c1 = ov1 > cmax1
            cvec1 = jnp.where(c1, v, cvec1)
            cmax1 = jnp.where(c1, ov1, cmax1)
        cm_v[g0, :] = cmax0
        ca_v[g0, :] = base + cvec0 * _L + lane
        cm_v[g1, :] = cmax1
        ca_v[g1, :] = base + cvec1 * _L + lane
        return _

    lax.fori_loop(0, n_gt // 2, g_step, None)

    for v in range(nvec):
        sl = pl.ds(v * _L, _L)
        ra = ra_v[sl]
        scx = plsc.load_gather(gcx_v, [ra])
        scy = plsc.load_gather(gcy_v, [ra])
        sw = plsc.load_gather(gw_v, [ra])
        sh = plsc.load_gather(gh_v, [ra])
        ax1 = av[0, sl]
        ay1 = av[1, sl]
        ax2 = av[2, sl]
        ay2 = av[3, sl]
        ew = ax2 - ax1 + 1.0
        eh = ay2 - ay1 + 1.0
        ecx = ax1 + 0.5 * ew
        ecy = ay1 + 0.5 * eh
        bb_v[0, sl] = (scx - ecx) / ew
        bb_v[1, sl] = (scy - ecy) / eh
        bb_v[2, sl] = sw / ew
        bb_v[3, sl] = sh / eh

    pltpu.sync_copy(rm_v, rowmax_hbm.at[wid])
    pltpu.sync_copy(bb_v, bb_hbm.at[wid])
    pltpu.sync_copy(cm_v, cmax_hbm.at[wid])
    pltpu.sync_copy(ca_v, carg_hbm.at[wid])


def _tc_merge_body(n_valid, g_valid, rm_ref, bb_ref, cm_ref, ca_ref,
                   a_ref, meta_ref, labels_ref, bbox_ref):
    cm = cm_ref[:, :]
    m = jnp.max(cm, axis=0, keepdims=True)
    cand = jnp.where(cm == m, ca_ref[:, :], jnp.int32(2 ** 30))
    carg_i = jnp.min(cand, axis=0, keepdims=True)

    n_pad = rm_ref.shape[0]
    g_pad = cm.shape[1]
    row_i = lax.broadcasted_iota(jnp.int32, (n_pad, g_pad), 0)
    pos_gt = jnp.any(row_i == carg_i, axis=1, keepdims=True)

    rm = rm_ref[:, :]
    labels = jnp.where(rm < _NEG_OV, 0.0, -1.0)
    labels = jnp.where(pos_gt, 1.0, labels)
    labels = jnp.where(rm >= _POS_OV, 1.0, labels)

    h = meta_ref[0, 0]
    w = meta_ref[0, 1]
    inside = ((a_ref[:, 0:1] >= 0.0) & (a_ref[:, 1:2] >= 0.0) &
              (a_ref[:, 2:3] < w) & (a_ref[:, 3:4] < h))
    labels_ref[:, :] = jnp.where(inside, labels, -1.0)

    bbox_ref[:, 0:2] = bb_ref[:, 0:2]
    bbox_ref[:, 2:3] = jnp.log(bb_ref[:, 2:3])
    bbox_ref[:, 3:4] = jnp.log(bb_ref[:, 3:4])


@functools.partial(jax.jit, static_argnums=(4, 5))
def _run(anchors_w, anchors_pad, gt_boxes, metadata, n_valid, g_valid):
    aw = anchors_w.shape[2]
    n_pad = _NW * aw
    g_pad = ((g_valid + _L - 1) // _L) * _L

    gtb = jnp.broadcast_to(gt_boxes[:, :, None], (g_valid, 4, _L))
    gp = jnp.zeros((4, g_pad), jnp.float32).at[:, :g_valid].set(gt_boxes.T)

    mesh = plsc.VectorSubcoreMesh(core_axis_name="c", subcore_axis_name="s",
                                  num_cores=_NC, num_subcores=_NS)
    sc = pl.kernel(
        functools.partial(_sc_body, aw, g_valid, g_pad),
        out_type=[
            jax.ShapeDtypeStruct((_NW, aw), jnp.float32),
            jax.ShapeDtypeStruct((_NW, 4, aw), jnp.float32),
            jax.ShapeDtypeStruct((_NW, g_valid, _L), jnp.float32),
            jax.ShapeDtypeStruct((_NW, g_valid, _L), jnp.int32),
        ],
        mesh=mesh,
        compiler_params=pltpu.CompilerParams(needs_layout_passes=False),
        scratch_types=[
            pltpu.VMEM((4, aw), jnp.float32),
            pltpu.VMEM((g_valid, 4, _L), jnp.float32),
            pltpu.VMEM((4, g_pad), jnp.float32),
            pltpu.VMEM((aw,), jnp.float32),
            pltpu.VMEM((aw,), jnp.float32),
            pltpu.VMEM((aw,), jnp.int32),
            pltpu.VMEM((g_valid, _L), jnp.float32),
            pltpu.VMEM((g_valid, _L), jnp.int32),
            pltpu.VMEM((g_pad,), jnp.float32),
            pltpu.VMEM((g_pad,), jnp.float32),
            pltpu.VMEM((g_pad,), jnp.float32),
            pltpu.VMEM((g_pad,), jnp.float32),
            pltpu.VMEM((g_pad,), jnp.float32),
            pltpu.VMEM((4, aw), jnp.float32),
        ],
    )
    rowmax, bb, cmax_w, carg_w = sc(anchors_w, gtb, gp)

    rowmax = rowmax.reshape(n_pad, 1)
    bb = jnp.transpose(bb, (0, 2, 1)).reshape(n_pad, 4)
    cmax_w = jnp.transpose(cmax_w, (0, 2, 1)).reshape(_NW * _L, g_valid)
    carg_w = jnp.transpose(carg_w, (0, 2, 1)).reshape(_NW * _L, g_valid)

    labels, bbox = pl.pallas_call(
        functools.partial(_tc_merge_body, n_valid, g_valid),
        out_shape=[
            jax.ShapeDtypeStruct((n_pad, 1), jnp.float32),
            jax.ShapeDtypeStruct((n_pad, 4), jnp.float32),
        ],
        in_specs=[
            pl.BlockSpec(memory_space=pltpu.VMEM),
            pl.BlockSpec(memory_space=pltpu.VMEM),
            pl.BlockSpec(memory_space=pltpu.VMEM),
            pl.BlockSpec(memory_space=pltpu.VMEM),
            pl.BlockSpec(memory_space=pltpu.VMEM),
            pl.BlockSpec(memory_space=pltpu.SMEM),
        ],
        out_specs=[
            pl.BlockSpec(memory_space=pltpu.VMEM),
            pl.BlockSpec(memory_space=pltpu.VMEM),
        ],
    )(rowmax, bb, cmax_w, carg_w, anchors_pad, metadata)
    return labels[:n_valid, 0], bbox[:n_valid, :]


def kernel(scores, gt_boxes, metadata):
    rr, cc = scores.shape[1], scores.shape[2]
    anchors_in = _inside_anchors((rr, cc), _STRIDE)
    n_valid = anchors_in.shape[0]
    aw = ((n_valid + _NW * _L - 1) // (_NW * _L)) * _L
    n_pad = _NW * aw
    pad = np.full((n_pad - n_valid, 4), -1.0e6, dtype=np.float32)
    pad[:, 2:] += 1.0
    a_full = np.concatenate([anchors_in, pad], axis=0)
    anchors_w = jnp.asarray(
        a_full.reshape(_NW, aw, 4).transpose(0, 2, 1).copy())
    anchors_pad = jnp.asarray(a_full)
    return _run(anchors_w, anchors_pad, gt_boxes, metadata,
                n_valid, gt_boxes.shape[0])
